# baseline (speedup 1.0000x reference)
import jax
import jax.numpy as jnp
from jax import lax
from jax.experimental import pallas as pl
from jax.experimental.pallas import tpu as pltpu
from jax.experimental.pallas import tpu_sc as plsc

TOKENS = 16384
NSEG = 16
D = 1024
NC = 2
NS = 16
LANES = 16

R_SC = 2048
NTILE = NC * NS
ROWS_PER_TILE = R_SC // NTILE
CHUNK = 32
NCHUNK = ROWS_PER_TILE // CHUNK
NGRP = D // (8 * LANES)
OUT_COLS = 128
NRED = D // OUT_COLS

BLK = 256
NBLK = (TOKENS - R_SC) // BLK


def _sc_body(flat_hbm, starts_hbm, ends_hbm, out_hbm,
             buf0, buf1, partial, buf2, out_buf, starts_v,
             ends_v, shared, sem0, sem1):
    c = lax.axis_index("c")
    s = lax.axis_index("s")
    w = s * NC + c
    row_base = w * ROWS_PER_TILE

    pltpu.sync_copy(starts_hbm, starts_v)
    pltpu.sync_copy(ends_hbm, ends_v)

    inf_v = jnp.full((LANES,), jnp.inf, jnp.float32)

    def init_body(j, carry):
        for seg in range(NSEG):
            partial[seg, pl.ds(j * LANES, LANES)] = inf_v
        return carry

    lax.fori_loop(0, D // LANES, init_body, 0)

    bufs = (buf0, buf1)
    sems = (sem0, sem1)

    def chunk_src(k):
        return flat_hbm.at[pl.ds(row_base + k * CHUNK, CHUNK), pl.ds(0, D)]

    for b in range(2):
        pltpu.async_copy(chunk_src(b), bufs[b], sems[b])

    def process(buf, k):
        g0 = row_base + k * CHUNK

        def seg_body(seg, carry):
            lo = starts_v[pl.ds(seg, 1)][0]
            hi = ends_v[pl.ds(seg, 1)][0]
            rs = jnp.clip(lo - g0, 0, CHUNK)
            re = jnp.clip(hi - g0, 0, CHUNK)

            @pl.when(rs < re)
            def _():
                for jg in range(NGRP):
                    base = jg * 8 * LANES
                    accs = tuple(partial[seg, pl.ds(base + jj * LANES, LANES)]
                                 for jj in range(8))

                    @plsc.parallel_loop(rs, re, carry=accs, unroll=4)
                    def accs(r, a):
                        return tuple(
                            jnp.minimum(
                                a[jj],
                                buf[r, pl.ds(base + jj * LANES, LANES)])
                            for jj in range(8))
                    for jj in range(8):
                        partial[seg, pl.ds(base + jj * LANES, LANES)] = accs[jj]

            return carry

        lax.fori_loop(0, NSEG, seg_body, 0)

    def outer(i, carry):
        k0 = i * 2
        for b in range(2):
            k = k0 + b
            pltpu.make_async_copy(chunk_src(k), bufs[b], sems[b]).wait()
            process(bufs[b], k)

            @pl.when(k + 2 < NCHUNK)
            def _():
                pltpu.async_copy(chunk_src(k + 2), bufs[b], sems[b])
        return carry

    lax.fori_loop(0, NCHUNK // 2, outer, 0)

    for rnd in range(2):
        @pl.when(jnp.logical_and(s >= rnd * 8, s < (rnd + 1) * 8))
        def _():
            pltpu.sync_copy(partial, shared.at[s - rnd * 8])

        plsc.subcore_barrier()

        @pl.when(s < NRED)
        def _():
            for t in range(8):
                pltpu.sync_copy(
                    shared.at[t, :, pl.ds(s * OUT_COLS, OUT_COLS)],
                    buf2.at[rnd * 8 + t])

        plsc.subcore_barrier()

    @pl.when(s < NRED)
    def _():
        nvec = OUT_COLS // LANES
        for seg in range(NSEG):
            accs = tuple(buf2[0, seg, pl.ds(jj * LANES, LANES)]
                         for jj in range(nvec))

            def red_body(t, a):
                return tuple(
                    jnp.minimum(a[jj], buf2[t, seg, pl.ds(jj * LANES, LANES)])
                    for jj in range(nvec))

            accs = lax.fori_loop(1, NS, red_body, accs)
            for jj in range(nvec):
                out_buf[seg, pl.ds(jj * LANES, LANES)] = accs[jj]

        pltpu.sync_copy(
            out_buf, out_hbm.at[c, :, pl.ds(s * OUT_COLS, OUT_COLS)])


def _sc_call(flat, starts, ends):
    mesh = plsc.VectorSubcoreMesh(core_axis_name="c", subcore_axis_name="s")
    f = pl.kernel(
        _sc_body,
        out_type=jax.ShapeDtypeStruct((NC, NSEG, D), jnp.float32),
        mesh=mesh,
        scratch_types=[
            pltpu.VMEM((CHUNK, D), jnp.float32),
            pltpu.VMEM((CHUNK, D), jnp.float32),
            pltpu.VMEM((NSEG, D), jnp.float32),
            pltpu.VMEM((NS, NSEG, OUT_COLS), jnp.float32),
            pltpu.VMEM((NSEG, OUT_COLS), jnp.float32),
            pltpu.VMEM((LANES,), jnp.int32),
            pltpu.VMEM((LANES,), jnp.int32),
            pltpu.VMEM_SHARED((NS // 2, NSEG, D), jnp.float32),
            pltpu.SemaphoreType.DMA,
            pltpu.SemaphoreType.DMA,
        ],
    )
    return f(flat, starts, ends)


def _tc_body(cu_ref, x_ref, o_ref, acc_ref):
    k = pl.program_id(0)

    @pl.when(k == 0)
    def _():
        acc_ref[...] = jnp.full((NSEG, D), jnp.inf, jnp.float32)

    g0 = R_SC + k * BLK
    g1 = g0 + BLK
    x = x_ref[...]
    bmin = jnp.min(x, axis=0, keepdims=True)
    rid = g0 + lax.broadcasted_iota(jnp.int32, (BLK, 1), 0)

    def seg_body(s, carry):
        lo = cu_ref[s]
        hi = cu_ref[s + 1]
        inter = jnp.logical_and(lo < g1, hi > g0)
        full = jnp.logical_and(lo <= g0, hi >= g1)

        @pl.when(jnp.logical_and(inter, full))
        def _():
            acc_ref[pl.ds(s, 1), :] = jnp.minimum(acc_ref[pl.ds(s, 1), :],
                                                  bmin)

        @pl.when(jnp.logical_and(inter, jnp.logical_not(full)))
        def _():
            m = jnp.logical_and(rid >= lo, rid < hi)
            colmin = jnp.min(jnp.where(m, x, jnp.inf), axis=0, keepdims=True)
            acc_ref[pl.ds(s, 1), :] = jnp.minimum(acc_ref[pl.ds(s, 1), :],
                                                  colmin)

        return carry

    lax.fori_loop(0, NSEG, seg_body, 0)

    @pl.when(k == NBLK - 1)
    def _():
        o_ref[...] = acc_ref[...]


def _tc_call(flat, cu_seqlens):
    grid_spec = pltpu.PrefetchScalarGridSpec(
        num_scalar_prefetch=1,
        grid=(NBLK,),
        in_specs=[pl.BlockSpec((BLK, D), lambda k, cu: (R_SC // BLK + k, 0))],
        out_specs=pl.BlockSpec((NSEG, D), lambda k, cu: (0, 0)),
        scratch_shapes=[pltpu.VMEM((NSEG, D), jnp.float32)],
    )
    return pl.pallas_call(
        _tc_body,
        grid_spec=grid_spec,
        out_shape=jax.ShapeDtypeStruct((NSEG, D), jnp.float32),
    )(cu_seqlens, flat)


def kernel(flat, cu_seqlens):
    starts = cu_seqlens[:NSEG]
    ends = cu_seqlens[1:NSEG + 1]
    out_sc = _sc_call(flat, starts, ends)
    out_tc = _tc_call(flat, cu_seqlens)
    return jnp.minimum(jnp.minimum(out_sc[0], out_sc[1]), out_tc)


# speedup vs baseline: 1.3473x; 1.3473x over previous
import jax
import jax.numpy as jnp
from jax import lax
from jax.experimental import pallas as pl
from jax.experimental.pallas import tpu as pltpu
from jax.experimental.pallas import tpu_sc as plsc

TOKENS = 16384
NSEG = 16
D = 1024
NC = 2
NS = 16
LANES = 16

R_SC = 8192
NTILE = NC * NS
ROWS_PER_TILE = R_SC // NTILE
CHUNK = 32
NCHUNK = ROWS_PER_TILE // CHUNK
NGRP = D // (8 * LANES)
OUT_COLS = 128
NRED = D // OUT_COLS

BLK = 512
NBLK = (TOKENS - R_SC) // BLK


def _sc_body(flat_hbm, starts_hbm, ends_hbm, out_hbm,
             buf0, buf1, partial, buf2, out_buf, starts_v,
             ends_v, shared, sem0, sem1):
    c = lax.axis_index("c")
    s = lax.axis_index("s")
    w = s * NC + c
    row_base = w * ROWS_PER_TILE

    pltpu.sync_copy(starts_hbm, starts_v)
    pltpu.sync_copy(ends_hbm, ends_v)

    inf_v = jnp.full((LANES,), jnp.inf, jnp.float32)

    def init_body(j, carry):
        for seg in range(NSEG):
            partial[seg, pl.ds(j * LANES, LANES)] = inf_v
        return carry

    lax.fori_loop(0, D // LANES, init_body, 0)

    bufs = (buf0, buf1)
    sems = (sem0, sem1)

    def chunk_src(k):
        return flat_hbm.at[pl.ds(row_base + k * CHUNK, CHUNK), pl.ds(0, D)]

    for b in range(2):
        pltpu.async_copy(chunk_src(b), bufs[b], sems[b])

    def process(buf, k):
        g0 = row_base + k * CHUNK

        def seg_body(seg, carry):
            lo = starts_v[pl.ds(seg, 1)][0]
            hi = ends_v[pl.ds(seg, 1)][0]
            rs = jnp.clip(lo - g0, 0, CHUNK)
            re = jnp.clip(hi - g0, 0, CHUNK)

            @pl.when(rs < re)
            def _():
                for jg in range(NGRP):
                    base = jg * 8 * LANES
                    accs = tuple(partial[seg, pl.ds(base + jj * LANES, LANES)]
                                 for jj in range(8))

                    @plsc.parallel_loop(rs, re, carry=accs, unroll=4)
                    def accs(r, a):
                        return tuple(
                            jnp.minimum(
                                a[jj],
                                buf[r, pl.ds(base + jj * LANES, LANES)])
                            for jj in range(8))
                    for jj in range(8):
                        partial[seg, pl.ds(base + jj * LANES, LANES)] = accs[jj]

            return carry

        lax.fori_loop(0, NSEG, seg_body, 0)

    def outer(i, carry):
        k0 = i * 2
        for b in range(2):
            k = k0 + b
            pltpu.make_async_copy(chunk_src(k), bufs[b], sems[b]).wait()
            process(bufs[b], k)

            @pl.when(k + 2 < NCHUNK)
            def _():
                pltpu.async_copy(chunk_src(k + 2), bufs[b], sems[b])
        return carry

    lax.fori_loop(0, NCHUNK // 2, outer, 0)

    for rnd in range(2):
        @pl.when(jnp.logical_and(s >= rnd * 8, s < (rnd + 1) * 8))
        def _():
            pltpu.sync_copy(partial, shared.at[s - rnd * 8])

        plsc.subcore_barrier()

        @pl.when(s < NRED)
        def _():
            pltpu.sync_copy(
                shared.at[:, :, pl.ds(s * OUT_COLS, OUT_COLS)],
                buf2.at[pl.ds(rnd * 8, 8)])

        plsc.subcore_barrier()

    @pl.when(s < NRED)
    def _():
        nvec = OUT_COLS // LANES
        for seg in range(NSEG):
            accs = tuple(buf2[0, seg, pl.ds(jj * LANES, LANES)]
                         for jj in range(nvec))

            def red_body(t, a):
                return tuple(
                    jnp.minimum(a[jj], buf2[t, seg, pl.ds(jj * LANES, LANES)])
                    for jj in range(nvec))

            accs = lax.fori_loop(1, NS, red_body, accs)
            for jj in range(nvec):
                out_buf[seg, pl.ds(jj * LANES, LANES)] = accs[jj]

        pltpu.sync_copy(
            out_buf, out_hbm.at[c, :, pl.ds(s * OUT_COLS, OUT_COLS)])


def _sc_call(flat, starts, ends):
    mesh = plsc.VectorSubcoreMesh(core_axis_name="c", subcore_axis_name="s")
    f = pl.kernel(
        _sc_body,
        out_type=jax.ShapeDtypeStruct((NC, NSEG, D), jnp.float32),
        mesh=mesh,
        scratch_types=[
            pltpu.VMEM((CHUNK, D), jnp.float32),
            pltpu.VMEM((CHUNK, D), jnp.float32),
            pltpu.VMEM((NSEG, D), jnp.float32),
            pltpu.VMEM((NS, NSEG, OUT_COLS), jnp.float32),
            pltpu.VMEM((NSEG, OUT_COLS), jnp.float32),
            pltpu.VMEM((LANES,), jnp.int32),
            pltpu.VMEM((LANES,), jnp.int32),
            pltpu.VMEM_SHARED((NS // 2, NSEG, D), jnp.float32),
            pltpu.SemaphoreType.DMA,
            pltpu.SemaphoreType.DMA,
        ],
    )
    return f(flat, starts, ends)


def _tc_body(cu_ref, first_ref, last_ref, x_ref, o_ref, acc_ref):
    k = pl.program_id(0)

    @pl.when(k == 0)
    def _():
        acc_ref[...] = jnp.full((NSEG, D), jnp.inf, jnp.float32)

    g0 = R_SC + k * BLK
    g1 = g0 + BLK
    x = x_ref[...]
    bmin = jnp.min(x, axis=0, keepdims=True)
    rid = g0 + lax.broadcasted_iota(jnp.int32, (BLK, 1), 0)

    def seg_body(s, carry):
        lo = cu_ref[s]
        hi = cu_ref[s + 1]
        inter = jnp.logical_and(lo < g1, hi > g0)
        full = jnp.logical_and(lo <= g0, hi >= g1)

        @pl.when(jnp.logical_and(inter, full))
        def _():
            acc_ref[pl.ds(s, 1), :] = jnp.minimum(acc_ref[pl.ds(s, 1), :],
                                                  bmin)

        @pl.when(jnp.logical_and(inter, jnp.logical_not(full)))
        def _():
            m = jnp.logical_and(rid >= lo, rid < hi)
            colmin = jnp.min(jnp.where(m, x, jnp.inf), axis=0, keepdims=True)
            acc_ref[pl.ds(s, 1), :] = jnp.minimum(acc_ref[pl.ds(s, 1), :],
                                                  colmin)

        return carry

    lax.fori_loop(first_ref[k], last_ref[k] + 1, seg_body, 0)

    @pl.when(k == NBLK - 1)
    def _():
        o_ref[...] = acc_ref[...]


def _tc_call(flat, cu_seqlens, starts, ends):
    g0s = R_SC + jnp.arange(NBLK, dtype=jnp.int32) * BLK
    firsts = jnp.searchsorted(ends, g0s, side="right").astype(jnp.int32)
    lasts = (jnp.searchsorted(starts, g0s + BLK, side="left") - 1).astype(
        jnp.int32)
    grid_spec = pltpu.PrefetchScalarGridSpec(
        num_scalar_prefetch=3,
        grid=(NBLK,),
        in_specs=[pl.BlockSpec((BLK, D),
                               lambda k, cu, fi, la: (R_SC // BLK + k, 0))],
        out_specs=pl.BlockSpec((NSEG, D), lambda k, cu, fi, la: (0, 0)),
        scratch_shapes=[pltpu.VMEM((NSEG, D), jnp.float32)],
    )
    return pl.pallas_call(
        _tc_body,
        grid_spec=grid_spec,
        out_shape=jax.ShapeDtypeStruct((NSEG, D), jnp.float32),
    )(cu_seqlens, firsts, lasts, flat)


def kernel(flat, cu_seqlens):
    starts = cu_seqlens[:NSEG]
    ends = cu_seqlens[1:NSEG + 1]
    out_sc = _sc_call(flat, starts, ends)
    out_tc = _tc_call(flat, cu_seqlens, starts, ends)
    return jnp.minimum(jnp.minimum(out_sc[0], out_sc[1]), out_tc)


# speedup vs baseline: 1.4576x; 1.0818x over previous
import jax
import jax.numpy as jnp
from jax import lax
from jax.experimental import pallas as pl
from jax.experimental.pallas import tpu as pltpu
from jax.experimental.pallas import tpu_sc as plsc

TOKENS = 16384
NSEG = 16
D = 1024
NC = 2
NS = 16
LANES = 16

R_SC = 7168
NTILE = NC * NS
ROWS_PER_TILE = R_SC // NTILE
CHUNK = 32
NCHUNK = ROWS_PER_TILE // CHUNK
NGRP = D // (8 * LANES)
OUT_COLS = 128
NRED = D // OUT_COLS

BLK = 512
NBLK = (TOKENS - R_SC) // BLK


def _sc_body(flat_hbm, starts_hbm, ends_hbm, out_hbm,
             buf0, buf1, partial, buf2, out_buf, starts_v,
             ends_v, shared, sem0, sem1):
    c = lax.axis_index("c")
    s = lax.axis_index("s")
    w = s * NC + c
    row_base = w * ROWS_PER_TILE

    pltpu.sync_copy(starts_hbm, starts_v)
    pltpu.sync_copy(ends_hbm, ends_v)

    inf_v = jnp.full((LANES,), jnp.inf, jnp.float32)

    def init_body(j, carry):
        for seg in range(NSEG):
            partial[seg, pl.ds(j * LANES, LANES)] = inf_v
        return carry

    lax.fori_loop(0, D // LANES, init_body, 0)

    bufs = (buf0, buf1)
    sems = (sem0, sem1)

    def chunk_src(k):
        return flat_hbm.at[pl.ds(row_base + k * CHUNK, CHUNK), pl.ds(0, D)]

    for b in range(2):
        pltpu.async_copy(chunk_src(b), bufs[b], sems[b])

    def process(buf, k):
        g0 = row_base + k * CHUNK

        def seg_body(seg, carry):
            lo = starts_v[pl.ds(seg, 1)][0]
            hi = ends_v[pl.ds(seg, 1)][0]
            rs = jnp.clip(lo - g0, 0, CHUNK)
            re = jnp.clip(hi - g0, 0, CHUNK)

            @pl.when(rs < re)
            def _():
                for jg in range(NGRP):
                    base = jg * 8 * LANES
                    accs = tuple(partial[seg, pl.ds(base + jj * LANES, LANES)]
                                 for jj in range(8))

                    @plsc.parallel_loop(rs, re, carry=accs, unroll=4)
                    def accs(r, a):
                        return tuple(
                            jnp.minimum(
                                a[jj],
                                buf[r, pl.ds(base + jj * LANES, LANES)])
                            for jj in range(8))
                    for jj in range(8):
                        partial[seg, pl.ds(base + jj * LANES, LANES)] = accs[jj]

            return carry

        lax.fori_loop(0, NSEG, seg_body, 0)

    def outer(i, carry):
        k0 = i * 2
        for b in range(2):
            k = k0 + b
            pltpu.make_async_copy(chunk_src(k), bufs[b], sems[b]).wait()
            process(bufs[b], k)

            @pl.when(k + 2 < NCHUNK)
            def _():
                pltpu.async_copy(chunk_src(k + 2), bufs[b], sems[b])
        return carry

    lax.fori_loop(0, NCHUNK // 2, outer, 0)

    for rnd in range(2):
        @pl.when(jnp.logical_and(s >= rnd * 8, s < (rnd + 1) * 8))
        def _():
            pltpu.sync_copy(partial, shared.at[s - rnd * 8])

        plsc.subcore_barrier()

        @pl.when(s < NRED)
        def _():
            pltpu.sync_copy(
                shared.at[:, :, pl.ds(s * OUT_COLS, OUT_COLS)],
                buf2.at[pl.ds(rnd * 8, 8)])

        plsc.subcore_barrier()

    @pl.when(s < NRED)
    def _():
        nvec = OUT_COLS // LANES
        for seg in range(NSEG):
            accs = tuple(buf2[0, seg, pl.ds(jj * LANES, LANES)]
                         for jj in range(nvec))

            def red_body(t, a):
                return tuple(
                    jnp.minimum(a[jj], buf2[t, seg, pl.ds(jj * LANES, LANES)])
                    for jj in range(nvec))

            accs = lax.fori_loop(1, NS, red_body, accs)
            for jj in range(nvec):
                out_buf[seg, pl.ds(jj * LANES, LANES)] = accs[jj]

        pltpu.sync_copy(
            out_buf, out_hbm.at[c, :, pl.ds(s * OUT_COLS, OUT_COLS)])


def _sc_call(flat, starts, ends):
    mesh = plsc.VectorSubcoreMesh(core_axis_name="c", subcore_axis_name="s")
    f = pl.kernel(
        _sc_body,
        out_type=jax.ShapeDtypeStruct((NC, NSEG, D), jnp.float32),
        mesh=mesh,
        scratch_types=[
            pltpu.VMEM((CHUNK, D), jnp.float32),
            pltpu.VMEM((CHUNK, D), jnp.float32),
            pltpu.VMEM((NSEG, D), jnp.float32),
            pltpu.VMEM((NS, NSEG, OUT_COLS), jnp.float32),
            pltpu.VMEM((NSEG, OUT_COLS), jnp.float32),
            pltpu.VMEM((LANES,), jnp.int32),
            pltpu.VMEM((LANES,), jnp.int32),
            pltpu.VMEM_SHARED((NS // 2, NSEG, D), jnp.float32),
            pltpu.SemaphoreType.DMA,
            pltpu.SemaphoreType.DMA,
        ],
    )
    return f(flat, starts, ends)


def _tc_body(cu_ref, first_ref, last_ref, x_ref, o_ref, acc_ref):
    k = pl.program_id(0)

    @pl.when(k == 0)
    def _():
        acc_ref[...] = jnp.full((NSEG, D), jnp.inf, jnp.float32)

    g0 = R_SC + k * BLK
    g1 = g0 + BLK
    x = x_ref[...]
    bmin = jnp.min(x, axis=0, keepdims=True)
    rid = g0 + lax.broadcasted_iota(jnp.int32, (BLK, 1), 0)

    def seg_body(s, carry):
        lo = cu_ref[s]
        hi = cu_ref[s + 1]
        inter = jnp.logical_and(lo < g1, hi > g0)
        full = jnp.logical_and(lo <= g0, hi >= g1)

        @pl.when(jnp.logical_and(inter, full))
        def _():
            acc_ref[pl.ds(s, 1), :] = jnp.minimum(acc_ref[pl.ds(s, 1), :],
                                                  bmin)

        @pl.when(jnp.logical_and(inter, jnp.logical_not(full)))
        def _():
            m = jnp.logical_and(rid >= lo, rid < hi)
            colmin = jnp.min(jnp.where(m, x, jnp.inf), axis=0, keepdims=True)
            acc_ref[pl.ds(s, 1), :] = jnp.minimum(acc_ref[pl.ds(s, 1), :],
                                                  colmin)

        return carry

    lax.fori_loop(first_ref[k], last_ref[k] + 1, seg_body, 0)

    @pl.when(k == NBLK - 1)
    def _():
        o_ref[...] = acc_ref[...]


def _tc_call(flat, cu_seqlens, starts, ends):
    g0s = R_SC + jnp.arange(NBLK, dtype=jnp.int32) * BLK
    firsts = jnp.sum((ends[None, :] <= g0s[:, None]).astype(jnp.int32),
                     axis=1)
    lasts = jnp.sum((starts[None, :] < (g0s + BLK)[:, None]).astype(jnp.int32),
                    axis=1) - 1
    grid_spec = pltpu.PrefetchScalarGridSpec(
        num_scalar_prefetch=3,
        grid=(NBLK,),
        in_specs=[pl.BlockSpec((BLK, D),
                               lambda k, cu, fi, la: (R_SC // BLK + k, 0))],
        out_specs=pl.BlockSpec((NSEG, D), lambda k, cu, fi, la: (0, 0)),
        scratch_shapes=[pltpu.VMEM((NSEG, D), jnp.float32)],
    )
    return pl.pallas_call(
        _tc_body,
        grid_spec=grid_spec,
        out_shape=jax.ShapeDtypeStruct((NSEG, D), jnp.float32),
    )(cu_seqlens, firsts, lasts, flat)


def kernel(flat, cu_seqlens):
    starts = cu_seqlens[:NSEG]
    ends = cu_seqlens[1:NSEG + 1]
    out_sc = _sc_call(flat, starts, ends)
    out_tc = _tc_call(flat, cu_seqlens, starts, ends)
    return jnp.minimum(jnp.minimum(out_sc[0], out_sc[1]), out_tc)
